# baseline (speedup 1.0000x reference)
import functools

import jax
import jax.numpy as jnp
from jax import lax
from jax.experimental import pallas as pl
from jax.experimental.pallas import tpu as pltpu
from jax.experimental.pallas import tpu_sc as plsc

N = 10000
NP = 10240
E = 320000
H = 64
G = 64

_NC = 2
_NS = 16
_NW = _NC * _NS
_K = 128
_C = 2
_NB = 3
_NCHUNK = 84
_NSUP = _NCHUNK // _C
_EPT = _K * _NCHUNK
_EPAD = _EPT * _NW
_RPT = NP // _NS


def _sc_agg_body(F, h_hbm, src_hbm, dst_hbm, w_hbm, zero_hbm,
                 outa_hbm, outb_hbm, src_v, dst_v, w_v,
                 buf0, buf1, buf2, agg_sh,
                 gsem0, gsem1, gsem2, ssem0, ssem1, ssem2):
    cid = lax.axis_index("c")
    sid = lax.axis_index("s")
    wid = sid * _NC + cid
    bufs = (buf0, buf1, buf2)
    gsems = (gsem0, gsem1, gsem2)
    ssems = (ssem0, ssem1, ssem2)

    pltpu.sync_copy(src_hbm.at[wid], src_v)
    pltpu.sync_copy(dst_hbm.at[wid], dst_v)
    pltpu.sync_copy(w_hbm.at[wid], w_v)
    r0 = sid * _RPT
    pltpu.sync_copy(zero_hbm.at[pl.ds(r0, _RPT)], agg_sh.at[pl.ds(r0, _RPT)])
    plsc.subcore_barrier()

    def scale(u, buf):
        for q in range(_C):
            def group(g, c, q=q):
                wrow = w_v[u * _C + q, pl.ds(g * 16, 16)]
                for e in range(16):
                    i = g * 16 + e
                    wv = jnp.full((16,), wrow[e], dtype=jnp.float32)
                    for f in range(F // 16):
                        buf[q * _K + i, pl.ds(f * 16, 16)] = (
                            buf[q * _K + i, pl.ds(f * 16, 16)] * wv)
                return c
            lax.fori_loop(0, _K // 16, group, 0)

    def start_gather(u, b):
        for q in range(_C):
            pltpu.async_copy(h_hbm.at[src_v.at[u * _C + q]],
                             bufs[b].at[pl.ds(q * _K, _K)], gsems[b])

    def wait_gather(u, b):
        pltpu.make_async_copy(h_hbm.at[pl.ds(0, _C * _K)], bufs[b],
                              gsems[b]).wait()

    def start_scatter(u, b):
        for q in range(_C):
            pltpu.async_copy(bufs[b].at[pl.ds(q * _K, _K)],
                             agg_sh.at[dst_v.at[u * _C + q]], ssems[b],
                             add=True)

    def wait_scatter(u, b):
        pltpu.make_async_copy(bufs[b], agg_sh.at[pl.ds(0, _C * _K)],
                              ssems[b]).wait()

    start_gather(0, 0)

    def tri(t, carry):
        for s in range(_NB):
            b, nb = s % _NB, (s + 1) % _NB
            u = t * _NB + s

            @pl.when(u >= 2)
            def _():
                wait_scatter(u - 2, nb)

            @pl.when(u + 1 < _NSUP)
            def _():
                start_gather(u + 1, nb)

            wait_gather(u, b)
            scale(u, bufs[b])
            start_scatter(u, b)
        return carry

    lax.fori_loop(0, _NSUP // _NB, tri, 0)
    wait_scatter(_NSUP - 2, (_NSUP - 2) % _NB)
    wait_scatter(_NSUP - 1, (_NSUP - 1) % _NB)
    plsc.subcore_barrier()

    @pl.when(cid == 0)
    def _():
        pltpu.sync_copy(agg_sh.at[pl.ds(r0, _RPT)], outa_hbm.at[pl.ds(r0, _RPT)])

    @pl.when(cid == 1)
    def _():
        pltpu.sync_copy(agg_sh.at[pl.ds(r0, _RPT)], outb_hbm.at[pl.ds(r0, _RPT)])


def _make_sc_agg(F):
    return functools.partial(
        pl.kernel,
        mesh=plsc.VectorSubcoreMesh(core_axis_name="c", subcore_axis_name="s"),
        compiler_params=pltpu.CompilerParams(use_tc_tiling_on_sc=False),
        out_type=[jax.ShapeDtypeStruct((NP, F), jnp.float32)] * 2,
        scratch_types=[
            pltpu.VMEM((_NCHUNK, _K), jnp.int32),
            pltpu.VMEM((_NCHUNK, _K), jnp.int32),
            pltpu.VMEM((_NCHUNK, _K), jnp.float32),
            pltpu.VMEM((_C * _K, F), jnp.float32),
            pltpu.VMEM((_C * _K, F), jnp.float32),
            pltpu.VMEM((_C * _K, F), jnp.float32),
            pltpu.VMEM_SHARED((NP, F), jnp.float32),
            pltpu.SemaphoreType.DMA,
            pltpu.SemaphoreType.DMA,
            pltpu.SemaphoreType.DMA,
            pltpu.SemaphoreType.DMA,
            pltpu.SemaphoreType.DMA,
            pltpu.SemaphoreType.DMA,
        ],
    )(functools.partial(_sc_agg_body, F))


_sc_agg64 = _make_sc_agg(64)


def _layer_body(aa_ref, ab_ref, h_ref, b_ref, wrel_ref, wroot_ref, out_ref):
    dn = (((1,), (1,)), ((), ()))
    agg = aa_ref[...] + ab_ref[...]
    rel = lax.dot_general(agg, wrel_ref[...], dn,
                          preferred_element_type=jnp.float32)
    root = lax.dot_general(h_ref[...], wroot_ref[...], dn,
                           preferred_element_type=jnp.float32)
    out_ref[...] = jnp.maximum(rel + b_ref[...] + root, 0.0)


def _layer(aa, ab, h, b, wrel, wroot):
    n, f = h.shape
    blk = 1024
    return pl.pallas_call(
        _layer_body,
        grid=(n // blk,),
        in_specs=[
            pl.BlockSpec((blk, f), lambda i: (i, 0)),
            pl.BlockSpec((blk, f), lambda i: (i, 0)),
            pl.BlockSpec((blk, f), lambda i: (i, 0)),
            pl.BlockSpec((1, H), lambda i: (0, 0)),
            pl.BlockSpec((H, f), lambda i: (0, 0)),
            pl.BlockSpec((H, f), lambda i: (0, 0)),
        ],
        out_specs=pl.BlockSpec((blk, H), lambda i: (i, 0)),
        out_shape=jax.ShapeDtypeStruct((n, H), jnp.float32),
    )(aa, ab, h, b, wrel, wroot)


def _final_body(aa_ref, ab_ref, h_ref, b_ref, wrel_ref, wroot_ref, bf_ref,
                wl1_ref, bl1_ref, wl2_ref, bl2_ref, out_ref, pooled):
    i = pl.program_id(0)

    @pl.when(i == 0)
    def _():
        pooled[...] = jnp.full((G, H), -jnp.inf, dtype=jnp.float32)

    dn = (((1,), (1,)), ((), ()))
    agg = aa_ref[...] + ab_ref[...]
    rel = lax.dot_general(agg, wrel_ref[...], dn,
                          preferred_element_type=jnp.float32)
    root = lax.dot_general(h_ref[...], wroot_ref[...], dn,
                           preferred_element_type=jnp.float32)
    h = jnp.maximum(rel + b_ref[...] + root, 0.0)
    bf = bf_ref[...]

    def g_body(g, carry):
        m = bf == g.astype(jnp.float32)
        col = jnp.max(jnp.where(m, h, -jnp.inf), axis=0, keepdims=True)
        pooled[pl.ds(g, 1), :] = jnp.maximum(pooled[pl.ds(g, 1), :], col)
        return carry

    lax.fori_loop(0, G, g_body, 0)

    @pl.when(i == pl.num_programs(0) - 1)
    def _():
        p = pooled[...]
        h1 = jnp.maximum(
            lax.dot_general(p, wl1_ref[...], dn,
                            preferred_element_type=jnp.float32) + bl1_ref[...],
            0.0)
        out_ref[...] = (lax.dot_general(h1, wl2_ref[...], dn,
                                        preferred_element_type=jnp.float32)
                        + bl2_ref[...])


def _final(aa, ab, h, b, wrel, wroot, bf, wl1, bl1, wl2, bl2):
    n, f = h.shape
    blk = 1024
    l1 = wl1.shape[0]
    return pl.pallas_call(
        _final_body,
        grid=(n // blk,),
        in_specs=[
            pl.BlockSpec((blk, f), lambda i: (i, 0)),
            pl.BlockSpec((blk, f), lambda i: (i, 0)),
            pl.BlockSpec((blk, f), lambda i: (i, 0)),
            pl.BlockSpec((1, H), lambda i: (0, 0)),
            pl.BlockSpec((H, f), lambda i: (0, 0)),
            pl.BlockSpec((H, f), lambda i: (0, 0)),
            pl.BlockSpec((blk, H), lambda i: (i, 0)),
            pl.BlockSpec((l1, H), lambda i: (0, 0)),
            pl.BlockSpec((1, l1), lambda i: (0, 0)),
            pl.BlockSpec((128, l1), lambda i: (0, 0)),
            pl.BlockSpec((G, 128), lambda i: (0, 0)),
        ],
        out_specs=pl.BlockSpec((G, 128), lambda i: (0, 0)),
        out_shape=jax.ShapeDtypeStruct((G, 128), jnp.float32),
        scratch_shapes=[pltpu.VMEM((G, H), jnp.float32)],
    )(aa, ab, h, b, wrel, wroot, bf, wl1, bl1, wl2, bl2)


def kernel(x, edge_index, batch, edge_weight,
           Wrel0, brel0, Wroot0, Wrel1, brel1, Wroot1, Wrel2, brel2, Wroot2,
           Wl1, bl1, Wl2, bl2):
    x_p = jnp.pad(x, ((0, NP - N), (0, 0)))
    src = jnp.pad(edge_index[0], (0, _EPAD - E)).reshape(_NW, _NCHUNK, _K)
    dst = jnp.pad(edge_index[1], (0, _EPAD - E)).reshape(_NW, _NCHUNK, _K)
    w = jnp.pad(edge_weight, (0, _EPAD - E)).reshape(_NW, _NCHUNK, _K)
    bf = jnp.pad(batch, (0, NP - N), constant_values=G)
    bf = jnp.broadcast_to(bf.astype(jnp.float32)[:, None], (NP, H))
    zeros64 = jnp.zeros((NP, H), jnp.float32)

    aaA, abA = _sc_agg64(x_p[:, :H], src, dst, w, zeros64)
    aaB, abB = _sc_agg64(x_p[:, H:], src, dst, w, zeros64)
    aa = jnp.concatenate([aaA, aaB], axis=1)
    ab = jnp.concatenate([abA, abB], axis=1)
    h1 = _layer(aa, ab, x_p, brel0.reshape(1, H), Wrel0, Wroot0)
    aa, ab = _sc_agg64(h1, src, dst, w, zeros64)
    h2 = _layer(aa, ab, h1, brel1.reshape(1, H), Wrel1, Wroot1)
    aa, ab = _sc_agg64(h2, src, dst, w, zeros64)
    wl2p = jnp.pad(Wl2, ((0, 127), (0, 0)))
    out128 = _final(aa, ab, h2, brel2.reshape(1, H), Wrel2, Wroot2, bf,
                    Wl1, bl1.reshape(1, -1), wl2p,
                    jnp.broadcast_to(bl2.reshape(1, 1), (G, 128)))
    return out128[:, :1]


# speedup vs baseline: 1.0050x; 1.0050x over previous
import functools

import jax
import jax.numpy as jnp
from jax import lax
from jax.experimental import pallas as pl
from jax.experimental.pallas import tpu as pltpu
from jax.experimental.pallas import tpu_sc as plsc

N = 10000
NP = 10240
E = 320000
H = 64
G = 64

_NC = 2
_NS = 16
_NW = _NC * _NS
_K = 128
_C = 1
_NB = 3
_NCHUNK = 84
_NSUP = _NCHUNK // _C
_EPT = _K * _NCHUNK
_EPAD = _EPT * _NW
_RPT = NP // _NS


def _sc_agg_body(F, h_hbm, src_hbm, dst_hbm, w_hbm, zero_hbm,
                 outa_hbm, outb_hbm, src_v, dst_v, w_v,
                 buf0, buf1, buf2, agg_sh,
                 gsem0, gsem1, gsem2, ssem0, ssem1, ssem2):
    cid = lax.axis_index("c")
    sid = lax.axis_index("s")
    wid = sid * _NC + cid
    bufs = (buf0, buf1, buf2)
    gsems = (gsem0, gsem1, gsem2)
    ssems = (ssem0, ssem1, ssem2)

    pltpu.sync_copy(src_hbm.at[wid], src_v)
    pltpu.sync_copy(dst_hbm.at[wid], dst_v)
    pltpu.sync_copy(w_hbm.at[wid], w_v)
    r0 = sid * _RPT
    pltpu.sync_copy(zero_hbm.at[pl.ds(r0, _RPT)], agg_sh.at[pl.ds(r0, _RPT)])
    plsc.subcore_barrier()

    def scale(u, buf):
        for q in range(_C):
            def group(g, c, q=q):
                wrow = w_v[u * _C + q, pl.ds(g * 16, 16)]
                for e in range(16):
                    i = g * 16 + e
                    wv = jnp.full((16,), wrow[e], dtype=jnp.float32)
                    for f in range(F // 16):
                        buf[q * _K + i, pl.ds(f * 16, 16)] = (
                            buf[q * _K + i, pl.ds(f * 16, 16)] * wv)
                return c
            lax.fori_loop(0, _K // 16, group, 0)

    def start_gather(u, b):
        for q in range(_C):
            pltpu.async_copy(h_hbm.at[src_v.at[u * _C + q]],
                             bufs[b].at[pl.ds(q * _K, _K)], gsems[b])

    def wait_gather(u, b):
        pltpu.make_async_copy(h_hbm.at[pl.ds(0, _C * _K)], bufs[b],
                              gsems[b]).wait()

    def start_scatter(u, b):
        for q in range(_C):
            pltpu.async_copy(bufs[b].at[pl.ds(q * _K, _K)],
                             agg_sh.at[dst_v.at[u * _C + q]], ssems[b],
                             add=True)

    def wait_scatter(u, b):
        pltpu.make_async_copy(bufs[b], agg_sh.at[pl.ds(0, _C * _K)],
                              ssems[b]).wait()

    start_gather(0, 0)

    def tri(t, carry):
        for s in range(_NB):
            b, nb = s % _NB, (s + 1) % _NB
            u = t * _NB + s

            @pl.when(u >= 2)
            def _():
                wait_scatter(u - 2, nb)

            @pl.when(u + 1 < _NSUP)
            def _():
                start_gather(u + 1, nb)

            wait_gather(u, b)
            scale(u, bufs[b])
            start_scatter(u, b)
        return carry

    lax.fori_loop(0, _NSUP // _NB, tri, 0)
    wait_scatter(_NSUP - 2, (_NSUP - 2) % _NB)
    wait_scatter(_NSUP - 1, (_NSUP - 1) % _NB)
    plsc.subcore_barrier()

    @pl.when(cid == 0)
    def _():
        pltpu.sync_copy(agg_sh.at[pl.ds(r0, _RPT)], outa_hbm.at[pl.ds(r0, _RPT)])

    @pl.when(cid == 1)
    def _():
        pltpu.sync_copy(agg_sh.at[pl.ds(r0, _RPT)], outb_hbm.at[pl.ds(r0, _RPT)])


def _make_sc_agg(F):
    return functools.partial(
        pl.kernel,
        mesh=plsc.VectorSubcoreMesh(core_axis_name="c", subcore_axis_name="s"),
        compiler_params=pltpu.CompilerParams(use_tc_tiling_on_sc=False),
        out_type=[jax.ShapeDtypeStruct((NP, F), jnp.float32)] * 2,
        scratch_types=[
            pltpu.VMEM((_NCHUNK, _K), jnp.int32),
            pltpu.VMEM((_NCHUNK, _K), jnp.int32),
            pltpu.VMEM((_NCHUNK, _K), jnp.float32),
            pltpu.VMEM((_C * _K, F), jnp.float32),
            pltpu.VMEM((_C * _K, F), jnp.float32),
            pltpu.VMEM((_C * _K, F), jnp.float32),
            pltpu.VMEM_SHARED((NP, F), jnp.float32),
            pltpu.SemaphoreType.DMA,
            pltpu.SemaphoreType.DMA,
            pltpu.SemaphoreType.DMA,
            pltpu.SemaphoreType.DMA,
            pltpu.SemaphoreType.DMA,
            pltpu.SemaphoreType.DMA,
        ],
    )(functools.partial(_sc_agg_body, F))


_sc_agg64 = _make_sc_agg(64)


def _layer_body(aa_ref, ab_ref, h_ref, b_ref, wrel_ref, wroot_ref, out_ref):
    dn = (((1,), (1,)), ((), ()))
    agg = aa_ref[...] + ab_ref[...]
    rel = lax.dot_general(agg, wrel_ref[...], dn,
                          preferred_element_type=jnp.float32)
    root = lax.dot_general(h_ref[...], wroot_ref[...], dn,
                           preferred_element_type=jnp.float32)
    out_ref[...] = jnp.maximum(rel + b_ref[...] + root, 0.0)


def _layer(aa, ab, h, b, wrel, wroot):
    n, f = h.shape
    blk = 1024
    return pl.pallas_call(
        _layer_body,
        grid=(n // blk,),
        in_specs=[
            pl.BlockSpec((blk, f), lambda i: (i, 0)),
            pl.BlockSpec((blk, f), lambda i: (i, 0)),
            pl.BlockSpec((blk, f), lambda i: (i, 0)),
            pl.BlockSpec((1, H), lambda i: (0, 0)),
            pl.BlockSpec((H, f), lambda i: (0, 0)),
            pl.BlockSpec((H, f), lambda i: (0, 0)),
        ],
        out_specs=pl.BlockSpec((blk, H), lambda i: (i, 0)),
        out_shape=jax.ShapeDtypeStruct((n, H), jnp.float32),
    )(aa, ab, h, b, wrel, wroot)


def _final_body(aa_ref, ab_ref, h_ref, b_ref, wrel_ref, wroot_ref, bf_ref,
                wl1_ref, bl1_ref, wl2_ref, bl2_ref, out_ref, pooled):
    i = pl.program_id(0)

    @pl.when(i == 0)
    def _():
        pooled[...] = jnp.full((G, H), -jnp.inf, dtype=jnp.float32)

    dn = (((1,), (1,)), ((), ()))
    agg = aa_ref[...] + ab_ref[...]
    rel = lax.dot_general(agg, wrel_ref[...], dn,
                          preferred_element_type=jnp.float32)
    root = lax.dot_general(h_ref[...], wroot_ref[...], dn,
                           preferred_element_type=jnp.float32)
    h = jnp.maximum(rel + b_ref[...] + root, 0.0)
    bf = bf_ref[...]

    def g_body(g, carry):
        m = bf == g.astype(jnp.float32)
        col = jnp.max(jnp.where(m, h, -jnp.inf), axis=0, keepdims=True)
        pooled[pl.ds(g, 1), :] = jnp.maximum(pooled[pl.ds(g, 1), :], col)
        return carry

    lax.fori_loop(0, G, g_body, 0)

    @pl.when(i == pl.num_programs(0) - 1)
    def _():
        p = pooled[...]
        h1 = jnp.maximum(
            lax.dot_general(p, wl1_ref[...], dn,
                            preferred_element_type=jnp.float32) + bl1_ref[...],
            0.0)
        out_ref[...] = (lax.dot_general(h1, wl2_ref[...], dn,
                                        preferred_element_type=jnp.float32)
                        + bl2_ref[...])


def _final(aa, ab, h, b, wrel, wroot, bf, wl1, bl1, wl2, bl2):
    n, f = h.shape
    blk = 1024
    l1 = wl1.shape[0]
    return pl.pallas_call(
        _final_body,
        grid=(n // blk,),
        in_specs=[
            pl.BlockSpec((blk, f), lambda i: (i, 0)),
            pl.BlockSpec((blk, f), lambda i: (i, 0)),
            pl.BlockSpec((blk, f), lambda i: (i, 0)),
            pl.BlockSpec((1, H), lambda i: (0, 0)),
            pl.BlockSpec((H, f), lambda i: (0, 0)),
            pl.BlockSpec((H, f), lambda i: (0, 0)),
            pl.BlockSpec((blk, H), lambda i: (i, 0)),
            pl.BlockSpec((l1, H), lambda i: (0, 0)),
            pl.BlockSpec((1, l1), lambda i: (0, 0)),
            pl.BlockSpec((128, l1), lambda i: (0, 0)),
            pl.BlockSpec((G, 128), lambda i: (0, 0)),
        ],
        out_specs=pl.BlockSpec((G, 128), lambda i: (0, 0)),
        out_shape=jax.ShapeDtypeStruct((G, 128), jnp.float32),
        scratch_shapes=[pltpu.VMEM((G, H), jnp.float32)],
    )(aa, ab, h, b, wrel, wroot, bf, wl1, bl1, wl2, bl2)


def kernel(x, edge_index, batch, edge_weight,
           Wrel0, brel0, Wroot0, Wrel1, brel1, Wroot1, Wrel2, brel2, Wroot2,
           Wl1, bl1, Wl2, bl2):
    x_p = jnp.pad(x, ((0, NP - N), (0, 0)))
    src = jnp.pad(edge_index[0], (0, _EPAD - E)).reshape(_NW, _NCHUNK, _K)
    dst = jnp.pad(edge_index[1], (0, _EPAD - E)).reshape(_NW, _NCHUNK, _K)
    w = jnp.pad(edge_weight, (0, _EPAD - E)).reshape(_NW, _NCHUNK, _K)
    bf = jnp.pad(batch, (0, NP - N), constant_values=G)
    bf = jnp.broadcast_to(bf.astype(jnp.float32)[:, None], (NP, H))
    zeros64 = jnp.zeros((NP, H), jnp.float32)

    aaA, abA = _sc_agg64(x_p[:, :H], src, dst, w, zeros64)
    aaB, abB = _sc_agg64(x_p[:, H:], src, dst, w, zeros64)
    aa = jnp.concatenate([aaA, aaB], axis=1)
    ab = jnp.concatenate([abA, abB], axis=1)
    h1 = _layer(aa, ab, x_p, brel0.reshape(1, H), Wrel0, Wroot0)
    aa, ab = _sc_agg64(h1, src, dst, w, zeros64)
    h2 = _layer(aa, ab, h1, brel1.reshape(1, H), Wrel1, Wroot1)
    aa, ab = _sc_agg64(h2, src, dst, w, zeros64)
    wl2p = jnp.pad(Wl2, ((0, 127), (0, 0)))
    out128 = _final(aa, ab, h2, brel2.reshape(1, H), Wrel2, Wroot2, bf,
                    Wl1, bl1.reshape(1, -1), wl2p,
                    jnp.broadcast_to(bl2.reshape(1, 1), (G, 128)))
    return out128[:, :1]


# speedup vs baseline: 2.3649x; 2.3530x over previous
import functools

import jax
import jax.numpy as jnp
from jax import lax
from jax.experimental import pallas as pl
from jax.experimental.pallas import tpu as pltpu
from jax.experimental.pallas import tpu_sc as plsc

N = 10000
NP = 10240
E = 320000
H = 64
G = 64

_NC = 2
_NS = 16
_NW = _NC * _NS
_K = 128
_C = 1
_NB = 3
_NCHUNK = 84
_NSUP = _NCHUNK // _C
_EPT = _K * _NCHUNK
_EPAD = _EPT * _NW
_RPT = NP // _NS


def _sc_agg_body(F, h_hbm, src_hbm, dst_hbm, w_hbm, zero_hbm,
                 outa_hbm, outb_hbm, src_v, dst_v, w_v,
                 buf0, buf1, buf2, agg_sh,
                 gsem0, gsem1, gsem2, ssem0, ssem1, ssem2):
    cid = lax.axis_index("c")
    sid = lax.axis_index("s")
    wid = sid * _NC + cid
    bufs = (buf0, buf1, buf2)
    gsems = (gsem0, gsem1, gsem2)
    ssems = (ssem0, ssem1, ssem2)

    pltpu.sync_copy(src_hbm.at[wid], src_v)
    pltpu.sync_copy(dst_hbm.at[wid], dst_v)
    pltpu.sync_copy(w_hbm.at[wid], w_v)
    r0 = sid * _RPT
    pltpu.sync_copy(zero_hbm.at[pl.ds(r0, _RPT)], agg_sh.at[pl.ds(r0, _RPT)])
    plsc.subcore_barrier()

    def scale(u, buf):
        for q in range(_C):
            def group(g, c, q=q):
                wrow = w_v[u * _C + q, pl.ds(g * 16, 16)]
                for e in range(16):
                    i = g * 16 + e
                    wv = jnp.full((16,), wrow[e], dtype=jnp.float32)
                    for f in range(F // 16):
                        buf[q * _K + i, pl.ds(f * 16, 16)] = (
                            buf[q * _K + i, pl.ds(f * 16, 16)] * wv)
                return c
            lax.fori_loop(0, _K // 16, group, 0)

    def start_gather(u, b):
        for q in range(_C):
            pltpu.async_copy(h_hbm.at[src_v.at[u * _C + q]],
                             bufs[b].at[pl.ds(q * _K, _K)], gsems[b])

    def wait_gather(u, b):
        pltpu.make_async_copy(h_hbm.at[pl.ds(0, _C * _K)], bufs[b],
                              gsems[b]).wait()

    def start_scatter(u, b):
        for q in range(_C):
            pltpu.async_copy(bufs[b].at[pl.ds(q * _K, _K)],
                             agg_sh.at[dst_v.at[u * _C + q]], ssems[b],
                             add=True)

    def wait_scatter(u, b):
        pltpu.make_async_copy(bufs[b], agg_sh.at[pl.ds(0, _C * _K)],
                              ssems[b]).wait()

    start_gather(0, 0)

    def tri(t, carry):
        for s in range(_NB):
            b, nb = s % _NB, (s + 1) % _NB
            u = t * _NB + s

            @pl.when(u >= 2)
            def _():
                wait_scatter(u - 2, nb)

            @pl.when(u + 1 < _NSUP)
            def _():
                start_gather(u + 1, nb)

            wait_gather(u, b)
            scale(u, bufs[b])
            start_scatter(u, b)
        return carry

    lax.fori_loop(0, _NSUP // _NB, tri, 0)
    wait_scatter(_NSUP - 2, (_NSUP - 2) % _NB)
    wait_scatter(_NSUP - 1, (_NSUP - 1) % _NB)
    plsc.subcore_barrier()

    @pl.when(cid == 0)
    def _():
        pltpu.sync_copy(agg_sh.at[pl.ds(r0, _RPT)], outa_hbm.at[pl.ds(r0, _RPT)])

    @pl.when(cid == 1)
    def _():
        pltpu.sync_copy(agg_sh.at[pl.ds(r0, _RPT)], outb_hbm.at[pl.ds(r0, _RPT)])


def _make_sc_agg(F):
    return functools.partial(
        pl.kernel,
        mesh=plsc.VectorSubcoreMesh(core_axis_name="c", subcore_axis_name="s"),
        compiler_params=pltpu.CompilerParams(use_tc_tiling_on_sc=False),
        out_type=[jax.ShapeDtypeStruct((NP, F), jnp.float32)] * 2,
        scratch_types=[
            pltpu.VMEM((_NCHUNK, _K), jnp.int32),
            pltpu.VMEM((_NCHUNK, _K), jnp.int32),
            pltpu.VMEM((_NCHUNK, _K), jnp.float32),
            pltpu.VMEM((_C * _K, F), jnp.float32),
            pltpu.VMEM((_C * _K, F), jnp.float32),
            pltpu.VMEM((_C * _K, F), jnp.float32),
            pltpu.VMEM_SHARED((NP, F), jnp.float32),
            pltpu.SemaphoreType.DMA,
            pltpu.SemaphoreType.DMA,
            pltpu.SemaphoreType.DMA,
            pltpu.SemaphoreType.DMA,
            pltpu.SemaphoreType.DMA,
            pltpu.SemaphoreType.DMA,
        ],
    )(functools.partial(_sc_agg_body, F))


_sc_agg64 = _make_sc_agg(64)


def _layer_body(aa_ref, ab_ref, h_ref, b_ref, wrel_ref, wroot_ref, out_ref):
    dn = (((1,), (1,)), ((), ()))
    agg = aa_ref[...] + ab_ref[...]
    rel = lax.dot_general(agg, wrel_ref[...], dn,
                          preferred_element_type=jnp.float32)
    root = lax.dot_general(h_ref[...], wroot_ref[...], dn,
                           preferred_element_type=jnp.float32)
    out_ref[...] = jnp.maximum(rel + b_ref[...] + root, 0.0)


def _layer(aa, ab, h, b, wrel, wroot):
    n, f = h.shape
    blk = 1024
    return pl.pallas_call(
        _layer_body,
        grid=(n // blk,),
        in_specs=[
            pl.BlockSpec((blk, f), lambda i: (i, 0)),
            pl.BlockSpec((blk, f), lambda i: (i, 0)),
            pl.BlockSpec((blk, f), lambda i: (i, 0)),
            pl.BlockSpec((1, H), lambda i: (0, 0)),
            pl.BlockSpec((H, f), lambda i: (0, 0)),
            pl.BlockSpec((H, f), lambda i: (0, 0)),
        ],
        out_specs=pl.BlockSpec((blk, H), lambda i: (i, 0)),
        out_shape=jax.ShapeDtypeStruct((n, H), jnp.float32),
    )(aa, ab, h, b, wrel, wroot)


def _final_body(aa_ref, ab_ref, h_ref, b_ref, wrel_ref, wroot_ref, bf_ref,
                wl1_ref, bl1_ref, wl2_ref, bl2_ref, out_ref, pooled):
    i = pl.program_id(0)

    @pl.when(i == 0)
    def _():
        pooled[...] = jnp.full((G, H), -jnp.inf, dtype=jnp.float32)

    dn = (((1,), (1,)), ((), ()))
    agg = aa_ref[...] + ab_ref[...]
    rel = lax.dot_general(agg, wrel_ref[...], dn,
                          preferred_element_type=jnp.float32)
    root = lax.dot_general(h_ref[...], wroot_ref[...], dn,
                           preferred_element_type=jnp.float32)
    h = jnp.maximum(rel + b_ref[...] + root, 0.0)
    bf = bf_ref[...]

    def g_body(g, carry):
        m = bf == g.astype(jnp.float32)
        col = jnp.max(jnp.where(m, h, -jnp.inf), axis=0, keepdims=True)
        pooled[pl.ds(g, 1), :] = jnp.maximum(pooled[pl.ds(g, 1), :], col)
        return carry

    lax.fori_loop(0, G, g_body, 0)

    @pl.when(i == pl.num_programs(0) - 1)
    def _():
        p = pooled[...]
        h1 = jnp.maximum(
            lax.dot_general(p, wl1_ref[...], dn,
                            preferred_element_type=jnp.float32) + bl1_ref[...],
            0.0)
        out_ref[...] = (lax.dot_general(h1, wl2_ref[...], dn,
                                        preferred_element_type=jnp.float32)
                        + bl2_ref[...])


def _final(aa, ab, h, b, wrel, wroot, bf, wl1, bl1, wl2, bl2):
    n, f = h.shape
    blk = 1024
    l1 = wl1.shape[0]
    return pl.pallas_call(
        _final_body,
        grid=(n // blk,),
        in_specs=[
            pl.BlockSpec((blk, f), lambda i: (i, 0)),
            pl.BlockSpec((blk, f), lambda i: (i, 0)),
            pl.BlockSpec((blk, f), lambda i: (i, 0)),
            pl.BlockSpec((1, H), lambda i: (0, 0)),
            pl.BlockSpec((H, f), lambda i: (0, 0)),
            pl.BlockSpec((H, f), lambda i: (0, 0)),
            pl.BlockSpec((blk, H), lambda i: (i, 0)),
            pl.BlockSpec((l1, H), lambda i: (0, 0)),
            pl.BlockSpec((1, l1), lambda i: (0, 0)),
            pl.BlockSpec((128, l1), lambda i: (0, 0)),
            pl.BlockSpec((G, 128), lambda i: (0, 0)),
        ],
        out_specs=pl.BlockSpec((G, 128), lambda i: (0, 0)),
        out_shape=jax.ShapeDtypeStruct((G, 128), jnp.float32),
        scratch_shapes=[pltpu.VMEM((G, H), jnp.float32)],
    )(aa, ab, h, b, wrel, wroot, bf, wl1, bl1, wl2, bl2)


def kernel(x, edge_index, batch, edge_weight,
           Wrel0, brel0, Wroot0, Wrel1, brel1, Wroot1, Wrel2, brel2, Wroot2,
           Wl1, bl1, Wl2, bl2):
    x_p = jnp.pad(x, ((0, NP - N), (0, 0)))
    pad_idx = jnp.arange(_EPAD - E, dtype=jnp.int32)
    pad_src = pad_idx % N
    pad_dst = N + (pad_idx % (NP - N))
    src = jnp.concatenate([edge_index[0], pad_src]).reshape(_NW, _NCHUNK, _K)
    dst = jnp.concatenate([edge_index[1], pad_dst]).reshape(_NW, _NCHUNK, _K)
    w = jnp.pad(edge_weight, (0, _EPAD - E)).reshape(_NW, _NCHUNK, _K)
    bf = jnp.pad(batch, (0, NP - N), constant_values=G)
    bf = jnp.broadcast_to(bf.astype(jnp.float32)[:, None], (NP, H))
    zeros64 = jnp.zeros((NP, H), jnp.float32)

    aaA, abA = _sc_agg64(x_p[:, :H], src, dst, w, zeros64)
    aaB, abB = _sc_agg64(x_p[:, H:], src, dst, w, zeros64)
    aa = jnp.concatenate([aaA, aaB], axis=1)
    ab = jnp.concatenate([abA, abB], axis=1)
    h1 = _layer(aa, ab, x_p, brel0.reshape(1, H), Wrel0, Wroot0)
    aa, ab = _sc_agg64(h1, src, dst, w, zeros64)
    h2 = _layer(aa, ab, h1, brel1.reshape(1, H), Wrel1, Wroot1)
    aa, ab = _sc_agg64(h2, src, dst, w, zeros64)
    wl2p = jnp.pad(Wl2, ((0, 127), (0, 0)))
    out128 = _final(aa, ab, h2, brel2.reshape(1, H), Wrel2, Wroot2, bf,
                    Wl1, bl1.reshape(1, -1), wl2p,
                    jnp.broadcast_to(bl2.reshape(1, 1), (G, 128)))
    return out128[:, :1]
